# Initial kernel scaffold; baseline (speedup 1.0000x reference)
import jax
import jax.numpy as jnp
from jax.experimental import pallas as pl


def kernel(x, aspect_ratio_ids, tile_indices, embedding, gate, tile_table):
    raise NotImplementedError("write your pallas kernel here")


# speedup vs baseline: 3.8623x; 3.8623x over previous
import jax
import jax.numpy as jnp
from jax.experimental import pallas as pl
from jax.experimental.pallas import tpu as pltpu

NUM_PATCHES = 1025
HIDDEN_DIM = 1280
MAX_NUM_TILES = 4


def _body(idx_ref, gate_ref, x_ref, emb_ref, tt_ref, o_ref):
    t = jnp.tanh(gate_ref[0])
    o_ref[...] = x_ref[...] + (1.0 - t) * emb_ref[...] + t * tt_ref[...]


def kernel(x, aspect_ratio_ids, tile_indices, embedding, gate, tile_table):
    bt = x.shape[0]
    idx = aspect_ratio_ids.astype(jnp.int32) * MAX_NUM_TILES + tile_indices.astype(jnp.int32)
    tt = tile_table.reshape(-1, NUM_PATCHES, HIDDEN_DIM)
    grid_spec = pltpu.PrefetchScalarGridSpec(
        num_scalar_prefetch=2,
        grid=(bt,),
        in_specs=[
            pl.BlockSpec((1, NUM_PATCHES, HIDDEN_DIM), lambda b, idx_ref, g_ref: (b, 0, 0)),
            pl.BlockSpec((NUM_PATCHES, HIDDEN_DIM), lambda b, idx_ref, g_ref: (0, 0)),
            pl.BlockSpec((1, NUM_PATCHES, HIDDEN_DIM), lambda b, idx_ref, g_ref: (idx_ref[b], 0, 0)),
        ],
        out_specs=pl.BlockSpec((1, NUM_PATCHES, HIDDEN_DIM), lambda b, idx_ref, g_ref: (b, 0, 0)),
    )
    return pl.pallas_call(
        _body,
        grid_spec=grid_spec,
        out_shape=jax.ShapeDtypeStruct(x.shape, x.dtype),
    )(idx, gate, x, embedding, tt)
